# baseline (speedup 1.0000x reference)
import functools

import jax
import jax.numpy as jnp
from jax import lax
from jax.experimental import pallas as pl
from jax.experimental.pallas import tpu as pltpu
from jax.experimental.pallas import tpu_sc as plsc

_BURST = 128


def _make_emb(B, V, D):
    info = plsc.get_sparse_core_info()
    nw = info.num_cores * info.num_subcores
    assert B % nw == 0
    b_per_w = B // nw
    n_bursts = b_per_w // _BURST

    mesh = plsc.VectorSubcoreMesh(core_axis_name="c", subcore_axis_name="s")

    @functools.partial(
        pl.kernel,
        mesh=mesh,
        out_type=jax.ShapeDtypeStruct((B, D), jnp.float32),
        scratch_types=[
            pltpu.VMEM((b_per_w,), jnp.int32),
            pltpu.VMEM((b_per_w, D), jnp.float32),
            pltpu.SemaphoreType.DMA,
            pltpu.SemaphoreType.DMA,
            pltpu.SemaphoreType.DMA,
            pltpu.SemaphoreType.DMA,
            pltpu.SemaphoreType.DMA,
            pltpu.SemaphoreType.DMA,
            pltpu.SemaphoreType.DMA,
            pltpu.SemaphoreType.DMA,
            pltpu.SemaphoreType.DMA,
        ],
        compiler_params=pltpu.CompilerParams(use_tc_tiling_on_sc=True),
    )
    def emb(
        idx_hbm, table_hbm, out_hbm, idx_v, rows_v,
        s0, s1, s2, s3, s4, s5, s6, s7, osem,
    ):
        wid = lax.axis_index("s") * info.num_cores + lax.axis_index("c")
        base = wid * b_per_w
        sems = [s0, s1, s2, s3, s4, s5, s6, s7]
        pltpu.sync_copy(idx_hbm.at[pl.ds(base, b_per_w)], idx_v)

        def burst(g, carry):
            c0 = g * _BURST
            for h in range(0, _BURST, 16):
                v16 = idx_v[pl.ds(c0 + h, 16)]
                for k in range(16):
                    s = v16[k]
                    pltpu.async_copy(
                        table_hbm.at[pl.ds(s, 1)],
                        rows_v.at[pl.ds(c0 + h + k, 1)],
                        sems[(h + k) % 8],
                    )
            @pl.when(g > 1)
            def _drain():
                for q in range(8):
                    pltpu.make_async_copy(
                        table_hbm.at[pl.ds(0, _BURST // 8)],
                        rows_v.at[pl.ds(0, _BURST // 8)],
                        sems[q],
                    ).wait()
                p0 = (g - 2) * _BURST
                pltpu.async_copy(
                    rows_v.at[pl.ds(p0, _BURST)],
                    out_hbm.at[pl.ds(base + p0, _BURST)],
                    osem,
                )
            return carry

        lax.fori_loop(0, n_bursts, burst, None, unroll=2)
        for _ in range(2):
            for q in range(8):
                pltpu.make_async_copy(
                    table_hbm.at[pl.ds(0, _BURST // 8)],
                    rows_v.at[pl.ds(0, _BURST // 8)],
                    sems[q],
                ).wait()
        for t in range(2):
            last = (n_bursts - 2 + t) * _BURST
            pltpu.async_copy(
                rows_v.at[pl.ds(last, _BURST)],
                out_hbm.at[pl.ds(base + last, _BURST)],
                osem,
            )
        for g in range(n_bursts):
            pltpu.make_async_copy(
                rows_v.at[pl.ds(0, _BURST)],
                out_hbm.at[pl.ds(base, _BURST)],
                osem,
            ).wait()

    return emb


def kernel(x, table):
    B = x.shape[0]
    V, D = table.shape
    emb = _make_emb(B, V, D)
    return emb(x.astype(jnp.int32), table)


# speedup vs baseline: 1.0088x; 1.0088x over previous
import functools

import jax
import jax.numpy as jnp
from jax import lax
from jax.experimental import pallas as pl
from jax.experimental.pallas import tpu as pltpu
from jax.experimental.pallas import tpu_sc as plsc

_BURST = 64


def _make_emb(B, V, D):
    info = plsc.get_sparse_core_info()
    nw = info.num_cores * info.num_subcores
    assert B % nw == 0
    b_per_w = B // nw
    n_bursts = b_per_w // _BURST

    mesh = plsc.VectorSubcoreMesh(core_axis_name="c", subcore_axis_name="s")

    @functools.partial(
        pl.kernel,
        mesh=mesh,
        out_type=jax.ShapeDtypeStruct((B, D), jnp.float32),
        scratch_types=[
            pltpu.VMEM((b_per_w,), jnp.int32),
            pltpu.VMEM((b_per_w, D), jnp.float32),
            pltpu.SemaphoreType.DMA,
            pltpu.SemaphoreType.DMA,
            pltpu.SemaphoreType.DMA,
            pltpu.SemaphoreType.DMA,
            pltpu.SemaphoreType.DMA,
            pltpu.SemaphoreType.DMA,
            pltpu.SemaphoreType.DMA,
            pltpu.SemaphoreType.DMA,
            pltpu.SemaphoreType.DMA,
        ],
        compiler_params=pltpu.CompilerParams(use_tc_tiling_on_sc=True),
    )
    def emb(
        idx_hbm, table_hbm, out_hbm, idx_v, rows_v,
        s0, s1, s2, s3, s4, s5, s6, s7, osem,
    ):
        wid = lax.axis_index("s") * info.num_cores + lax.axis_index("c")
        base = wid * b_per_w
        sems = [s0, s1, s2, s3, s4, s5, s6, s7]
        pltpu.sync_copy(idx_hbm.at[pl.ds(base, b_per_w)], idx_v)

        def burst(g, carry):
            c0 = g * _BURST
            for h in range(0, _BURST, 16):
                v16 = idx_v[pl.ds(c0 + h, 16)]
                for k in range(16):
                    s = v16[k]
                    pltpu.async_copy(
                        table_hbm.at[pl.ds(s, 1)],
                        rows_v.at[pl.ds(c0 + h + k, 1)],
                        sems[(h + k) % 8],
                    )
            @pl.when(g > 1)
            def _drain():
                for q in range(8):
                    pltpu.make_async_copy(
                        table_hbm.at[pl.ds(0, _BURST // 8)],
                        rows_v.at[pl.ds(0, _BURST // 8)],
                        sems[q],
                    ).wait()
                p0 = (g - 2) * _BURST
                pltpu.async_copy(
                    rows_v.at[pl.ds(p0, _BURST)],
                    out_hbm.at[pl.ds(base + p0, _BURST)],
                    osem,
                )
            return carry

        lax.fori_loop(0, n_bursts, burst, None, unroll=2)
        for _ in range(2):
            for q in range(8):
                pltpu.make_async_copy(
                    table_hbm.at[pl.ds(0, _BURST // 8)],
                    rows_v.at[pl.ds(0, _BURST // 8)],
                    sems[q],
                ).wait()
        for t in range(2):
            last = (n_bursts - 2 + t) * _BURST
            pltpu.async_copy(
                rows_v.at[pl.ds(last, _BURST)],
                out_hbm.at[pl.ds(base + last, _BURST)],
                osem,
            )
        for g in range(n_bursts):
            pltpu.make_async_copy(
                rows_v.at[pl.ds(0, _BURST)],
                out_hbm.at[pl.ds(base, _BURST)],
                osem,
            ).wait()

    return emb


def kernel(x, table):
    B = x.shape[0]
    V, D = table.shape
    emb = _make_emb(B, V, D)
    return emb(x.astype(jnp.int32), table)
